# Initial kernel scaffold; baseline (speedup 1.0000x reference)
import jax
import jax.numpy as jnp
from jax.experimental import pallas as pl


def kernel(pred_logits, pred_boxes, target_sizes):
    raise NotImplementedError("write your pallas kernel here")


# speedup vs baseline: 2.8287x; 2.8287x over previous
import functools

import jax
import jax.numpy as jnp
from jax import lax
from jax.experimental import pallas as pl
from jax.experimental.pallas import tpu as pltpu
from jax.experimental.pallas import tpu_sc as plsc

BS_, Q_, C_ = 128, 300, 92
N_ = Q_ * C_
NV_ = N_ // 16
K_ = Q_
CAND_ = 320
IMGS_PER_W_ = BS_ // 32
INT_MIN_ = -2147483648


def _sc_body(lg_hbm, pb_hbm, ts_hbm, outs_hbm, outl_hbm, outb_hbm,
             lgv, kv, hist, candk, candi, rankv, pbv, tsv, outsv, outlv,
             outbv):
    lanes = jax.lax.iota(jnp.int32, 16)
    ones = jnp.ones((16,), jnp.int32)
    wid = lax.axis_index("s") * 2 + lax.axis_index("c")

    pltpu.sync_copy(ts_hbm, tsv)

    def per_image(t, _):
        b = wid * IMGS_PER_W_ + t
        pltpu.sync_copy(lg_hbm.at[b], lgv)
        pltpu.sync_copy(pb_hbm.at[b], pbv)

        def zero_hist(i, _):
            hist[pl.ds(i * 16, 16)] = jnp.zeros((16,), jnp.int32)
            return 0

        lax.fori_loop(0, 256, zero_hist, 0)

        def pass0(jv, _):
            x = lgv[pl.ds(jv * 16, 16)]
            u = lax.bitcast_convert_type(x, jnp.int32)
            k = u ^ ((u >> 31) & jnp.int32(0x7FFFFFFF))
            kv[pl.ds(jv * 16, 16)] = k
            d0 = ((k >> 24) & 0xFF) ^ 0x80
            plsc.addupdate_scatter(hist, [lanes * 256 + d0], ones)
            return 0

        lax.fori_loop(0, NV_, pass0, 0)

        def scan_level(r_need):
            suff = []
            blocks = []
            for j in range(16):
                tot = hist[pl.ds(j * 16, 16)]
                for l in range(1, 16):
                    tot = tot + hist[pl.ds(l * 256 + j * 16, 16)]
                rc = lax.rev(plsc.cumsum(lax.rev(tot, (0,))), (0,))
                suff.append(rc)
                blocks.append(jnp.sum(tot))
            d_cnt = jnp.int32(0)
            g_above = jnp.int32(0)
            carry = jnp.int32(0)
            for j in range(15, -1, -1):
                s = suff[j] + carry
                carry = carry + blocks[j]
                d_cnt = d_cnt + jnp.sum((s >= r_need).astype(jnp.int32))
                g_above = jnp.maximum(
                    g_above, jnp.max(jnp.where(s < r_need, s, 0)))
            return d_cnt - 1, g_above

        d0_bucket, g0 = scan_level(jnp.int32(K_))
        prefix = d0_bucket ^ 0x80
        g_total = g0
        r_need = jnp.int32(K_) - g_total

        for level in (1, 2, 3):
            lax.fori_loop(0, 256, zero_hist, 0)
            pm_shift = 32 - 8 * level
            pm_mask = jnp.int32((1 << (8 * level)) - 1)
            d_shift = 24 - 8 * level
            pfx = prefix

            def passL(jv, _, pm_shift=pm_shift, pm_mask=pm_mask,
                      d_shift=d_shift, pfx=pfx):
                k = kv[pl.ds(jv * 16, 16)]
                match = ((k >> pm_shift) & pm_mask) == pfx
                d = (k >> d_shift) & 0xFF
                plsc.addupdate_scatter(hist, [lanes * 256 + d], ones,
                                       mask=match)
                return 0

            lax.fori_loop(0, NV_, passL, 0)
            d_bucket, g_above = scan_level(r_need)
            prefix = (prefix << 8) | d_bucket
            g_total = g_total + g_above
            r_need = jnp.int32(K_) - g_total

        thr = prefix
        m_eq = r_need

        def compact(jv, carry):
            off, eqc = carry
            k = kv[pl.ds(jv * 16, 16)]
            gt = k > thr
            eq = k == thr
            eq_run = plsc.cumsum(eq.astype(jnp.int32))
            take_eq = eq & ((eqc + eq_run) <= m_eq)
            sel = gt | take_eq
            seli = sel.astype(jnp.int32)
            pos = off + plsc.cumsum(seli) - 1
            plsc.store_scatter(candk, [pos], k, mask=sel)
            plsc.store_scatter(candi, [pos], jv * 16 + lanes, mask=sel)
            return off + jnp.sum(seli), eqc + jnp.sum(take_eq.astype(jnp.int32))

        lax.fori_loop(0, NV_, compact, (jnp.int32(0), jnp.int32(0)))

        pad_i = jnp.full((16,), 0x3FFFFFFF, dtype=jnp.int32)
        pad_k = jnp.full((16,), INT_MIN_, dtype=jnp.int32)
        plsc.store_scatter(candk, [300 + lanes], pad_k)
        plsc.store_scatter(candk, [304 + lanes], pad_k)
        plsc.store_scatter(candi, [300 + lanes], pad_i)
        plsc.store_scatter(candi, [304 + lanes], pad_i)

        keys = [candk[pl.ds(i * 16, 16)] for i in range(20)]
        posv = [lanes + 16 * i for i in range(20)]

        def rank_step(j, accs):
            kj = plsc.load_gather(candk, [jnp.full((16,), j, jnp.int32)])
            out = []
            for i in range(20):
                gtc = (keys[i] < kj).astype(jnp.int32)
                tie = ((keys[i] == kj) & (j < posv[i])).astype(jnp.int32)
                out.append(accs[i] + gtc + tie)
            return tuple(out)

        accs = lax.fori_loop(0, CAND_, rank_step,
                             tuple(jnp.zeros((16,), jnp.int32)
                                   for _ in range(20)))
        for i in range(20):
            rankv[pl.ds(i * 16, 16)] = accs[i]

        w_scale = plsc.load_gather(
            tsv, [jnp.full((16,), 2 * b + 1, jnp.int32)]).astype(jnp.float32)
        h_scale = plsc.load_gather(
            tsv, [jnp.full((16,), 2 * b, jnp.int32)]).astype(jnp.float32)

        def emit(jv, _):
            rk = rankv[pl.ds(jv * 16, 16)]
            sel = rk < K_
            k = candk[pl.ds(jv * 16, 16)]
            ci = candi[pl.ds(jv * 16, 16)]
            x = lax.bitcast_convert_type(k ^ ((k >> 31) & jnp.int32(0x7FFFFFFF)),
                                         jnp.float32)
            score = 1.0 / (1.0 + jnp.exp(-x))
            lab = (ci % C_).astype(jnp.float32)
            q = ci // C_
            cx = plsc.load_gather(pbv, [q * 4], mask=sel)
            cy = plsc.load_gather(pbv, [q * 4 + 1], mask=sel)
            w = plsc.load_gather(pbv, [q * 4 + 2], mask=sel)
            h = plsc.load_gather(pbv, [q * 4 + 3], mask=sel)
            x1 = (cx - 0.5 * w) * w_scale
            y1 = (cy - 0.5 * h) * h_scale
            x2 = (cx + 0.5 * w) * w_scale
            y2 = (cy + 0.5 * h) * h_scale
            plsc.store_scatter(outsv, [rk], score, mask=sel)
            plsc.store_scatter(outlv, [rk], lab, mask=sel)
            plsc.store_scatter(outbv, [rk * 4], x1, mask=sel)
            plsc.store_scatter(outbv, [rk * 4 + 1], y1, mask=sel)
            plsc.store_scatter(outbv, [rk * 4 + 2], x2, mask=sel)
            plsc.store_scatter(outbv, [rk * 4 + 3], y2, mask=sel)
            return 0

        lax.fori_loop(0, 20, emit, 0)

        pltpu.sync_copy(outsv, outs_hbm.at[b])
        pltpu.sync_copy(outlv, outl_hbm.at[b])
        pltpu.sync_copy(outbv, outb_hbm.at[b])
        return 0

    lax.fori_loop(0, IMGS_PER_W_, per_image, 0)


@jax.jit
def kernel(pred_logits, pred_boxes, target_sizes):
    lg = pred_logits.reshape(BS_, N_)
    pb = pred_boxes.reshape(BS_, Q_ * 4)
    ts = target_sizes.reshape(BS_ * 2)

    f32 = jnp.float32
    run = pl.kernel(
        _sc_body,
        out_type=[
            jax.ShapeDtypeStruct((BS_, K_), f32),
            jax.ShapeDtypeStruct((BS_, K_), f32),
            jax.ShapeDtypeStruct((BS_, K_ * 4), f32),
        ],
        mesh=plsc.VectorSubcoreMesh(core_axis_name="c",
                                    subcore_axis_name="s"),
        compiler_params=pltpu.CompilerParams(needs_layout_passes=False),
        scratch_types=[
            pltpu.VMEM((N_,), f32),
            pltpu.VMEM((N_,), jnp.int32),
            pltpu.VMEM((4096,), jnp.int32),
            pltpu.VMEM((CAND_,), jnp.int32),
            pltpu.VMEM((CAND_,), jnp.int32),
            pltpu.VMEM((CAND_,), jnp.int32),
            pltpu.VMEM((Q_ * 4,), f32),
            pltpu.VMEM((BS_ * 2,), jnp.int32),
            pltpu.VMEM((K_,), f32),
            pltpu.VMEM((K_,), f32),
            pltpu.VMEM((K_ * 4,), f32),
        ],
    )
    scores, labels, boxes = run(lg, pb, ts)
    return scores, labels, boxes.reshape(BS_, K_, 4)
